# initial kernel scaffold (unmeasured)
import jax
import jax.numpy as jnp
from jax import lax
from jax.experimental import pallas as pl
from jax.experimental.pallas import tpu as pltpu

B, H, D, BS = 8, 8, 128, 16
NB_LOCAL = 512
NTOK = NB_LOCAL * BS
SCALE = D ** -0.5
NEG = -1e30


def kernel(Q, K, V, bt, lens):
    my_x = lax.axis_index("x")

    page_ids = my_x * NB_LOCAL + jnp.arange(NB_LOCAL, dtype=jnp.int32)
    valid = jnp.arange(bt.shape[1], dtype=jnp.int32)[None, :] < lens[:, None]
    cnt = jnp.sum(
        (bt[:, :, None] == page_ids[None, None, :]) & valid[:, :, None],
        axis=1,
        dtype=jnp.int32,
    )
    w = jnp.repeat(cnt.astype(jnp.float32), BS, axis=1)

    def body(q_ref, k_ref, v_ref, w_ref, out_ref,
             m_acc, l_acc, o_acc, send_buf, recv_buf, send_sem, recv_sem):
        h = pl.program_id(0)

        qh = q_ref[...].reshape(B, D)
        kh = k_ref[...].reshape(NTOK, D)
        vh = v_ref[...].reshape(NTOK, D)
        wv = w_ref[...]

        s = lax.dot_general(
            qh, kh, (((1,), (1,)), ((), ())),
            preferred_element_type=jnp.float32,
        ) * SCALE
        s = jnp.where(wv > 0.0, s, NEG)
        m = jnp.max(s, axis=1, keepdims=True)
        p = wv * jnp.exp(s - m)
        l = jnp.sum(p, axis=1, keepdims=True)
        o = lax.dot_general(
            p, vh, (((1,), (0,)), ((), ())),
            preferred_element_type=jnp.float32,
        )

        m_acc[:, pl.ds(h, 1)] = m
        l_acc[:, pl.ds(h, 1)] = l
        o_acc[:, pl.ds(h, 1), :] = o[:, None, :]

        @pl.when(h == H - 1)
        def _():
            x = lax.axis_index("x")
            y = lax.axis_index("y")
            z = lax.axis_index("z")
            partner = (1 - x, y, z)

            send_buf[0, :, :] = o_acc[...].reshape(B * H, D)
            send_buf[1, :, 0:1] = m_acc[...].reshape(B * H, 1)
            send_buf[2, :, 0:1] = l_acc[...].reshape(B * H, 1)

            bsem = pltpu.get_barrier_semaphore()
            pl.semaphore_signal(
                bsem, inc=1,
                device_id=partner, device_id_type=pl.DeviceIdType.MESH,
            )
            pl.semaphore_wait(bsem, 1)

            rdma = pltpu.make_async_remote_copy(
                src_ref=send_buf,
                dst_ref=recv_buf,
                send_sem=send_sem,
                recv_sem=recv_sem,
                device_id=partner,
                device_id_type=pl.DeviceIdType.MESH,
            )
            rdma.start()
            rdma.wait()

            o_l = send_buf[0, :, :]
            m_l = send_buf[1, :, 0:1]
            l_l = send_buf[2, :, 0:1]
            o_r = recv_buf[0, :, :]
            m_r = recv_buf[1, :, 0:1]
            l_r = recv_buf[2, :, 0:1]

            mx = jnp.maximum(m_l, m_r)
            a = jnp.exp(m_l - mx)
            b = jnp.exp(m_r - mx)
            lsum = a * l_l + b * l_r
            out = (a * o_l + b * o_r) / lsum
            out_ref[...] = out.reshape(B, 1, H, D)

    return pl.pallas_call(
        body,
        grid=(H,),
        out_shape=jax.ShapeDtypeStruct((B, 1, H, D), jnp.float32),
        in_specs=[
            pl.BlockSpec((B, 1, 1, D), lambda h: (0, 0, h, 0)),
            pl.BlockSpec((NB_LOCAL, BS, 1, D), lambda h: (0, 0, h, 0)),
            pl.BlockSpec((NB_LOCAL, BS, 1, D), lambda h: (0, 0, h, 0)),
            pl.BlockSpec((B, NTOK), lambda h: (0, 0)),
        ],
        out_specs=pl.BlockSpec((B, 1, H, D), lambda h: (0, 0, 0, 0)),
        scratch_shapes=[
            pltpu.VMEM((B, H), jnp.float32),
            pltpu.VMEM((B, H), jnp.float32),
            pltpu.VMEM((B, H, D), jnp.float32),
            pltpu.VMEM((3, B * H, D), jnp.float32),
            pltpu.VMEM((3, B * H, D), jnp.float32),
            pltpu.SemaphoreType.DMA,
            pltpu.SemaphoreType.DMA,
        ],
        compiler_params=pltpu.CompilerParams(
            dimension_semantics=("arbitrary",),
            collective_id=0,
        ),
    )(Q, K, V, w)


# baseline (device time: 59965 ns/iter reference)
import jax
import jax.numpy as jnp
from jax import lax
from jax.experimental import pallas as pl
from jax.experimental.pallas import tpu as pltpu

B, H, D, BS = 8, 8, 128, 16
NB_LOCAL = 512
NTOK = NB_LOCAL * BS
CHUNK = 64
NCHUNK = NB_LOCAL // CHUNK
CTOK = CHUNK * BS
SCALE = D ** -0.5
NEG = -1e30


def kernel(Q, K, V, bt, lens):
    my_x = lax.axis_index("x")

    page_ids = my_x * NB_LOCAL + jnp.arange(NB_LOCAL, dtype=jnp.int32)
    valid = jnp.arange(bt.shape[1], dtype=jnp.int32)[None, :] < lens[:, None]
    cnt = jnp.sum(
        (bt[:, :, None] == page_ids[None, None, :]) & valid[:, :, None],
        axis=1,
        dtype=jnp.int32,
    )
    w = jnp.repeat(cnt.astype(jnp.float32), BS, axis=1)

    def body(q_ref, k_ref, v_ref, w_ref, out_ref,
             m_acc, l_acc, o_acc, recv_m, recv_l, recv_o,
             send_sems, recv_sems):
        c = pl.program_id(0)

        @pl.when(c == 0)
        def _():
            m_acc[...] = jnp.full((B, H), NEG, jnp.float32)
            l_acc[...] = jnp.zeros((B, H), jnp.float32)
            o_acc[...] = jnp.zeros((B, H, D), jnp.float32)

        wv = w_ref[...]
        for h in range(H):
            qh = q_ref[:, 0, h, :]
            kh = k_ref[:, :, h, :].reshape(CTOK, D)
            vh = v_ref[:, :, h, :].reshape(CTOK, D)

            s = lax.dot_general(
                qh, kh, (((1,), (1,)), ((), ())),
                preferred_element_type=jnp.float32,
            ) * SCALE
            s = jnp.where(wv > 0.0, s, NEG)

            m_old = m_acc[:, h:h + 1]
            m_new = jnp.maximum(m_old, jnp.max(s, axis=1, keepdims=True))
            alpha = jnp.exp(m_old - m_new)
            p = wv * jnp.exp(s - m_new)
            l_new = alpha * l_acc[:, h:h + 1] + jnp.sum(p, axis=1, keepdims=True)
            pv = lax.dot_general(
                p, vh, (((1,), (0,)), ((), ())),
                preferred_element_type=jnp.float32,
            )
            o_acc[:, h, :] = alpha * o_acc[:, h, :] + pv
            m_acc[:, h:h + 1] = m_new
            l_acc[:, h:h + 1] = l_new

        @pl.when(c == NCHUNK - 1)
        def _():
            x = lax.axis_index("x")
            y = lax.axis_index("y")
            z = lax.axis_index("z")
            partner = (1 - x, y, z)

            bsem = pltpu.get_barrier_semaphore()
            pl.semaphore_signal(
                bsem, inc=1,
                device_id=partner, device_id_type=pl.DeviceIdType.MESH,
            )
            pl.semaphore_wait(bsem, 1)

            rdmas = [
                pltpu.make_async_remote_copy(
                    src_ref=src, dst_ref=dst,
                    send_sem=send_sems.at[i], recv_sem=recv_sems.at[i],
                    device_id=partner, device_id_type=pl.DeviceIdType.MESH,
                )
                for i, (src, dst) in enumerate(
                    [(m_acc, recv_m), (l_acc, recv_l), (o_acc, recv_o)]
                )
            ]
            for r in rdmas:
                r.start()
            for r in rdmas:
                r.wait()

            m_l, l_l, o_l = m_acc[...], l_acc[...], o_acc[...]
            m_r, l_r, o_r = recv_m[...], recv_l[...], recv_o[...]

            mx = jnp.maximum(m_l, m_r)
            a = jnp.exp(m_l - mx)
            b = jnp.exp(m_r - mx)
            lsum = a * l_l + b * l_r
            out = (a[:, :, None] * o_l + b[:, :, None] * o_r) / lsum[:, :, None]
            out_ref[:, 0, :, :] = out

    return pl.pallas_call(
        body,
        grid=(NCHUNK,),
        out_shape=jax.ShapeDtypeStruct((B, 1, H, D), jnp.float32),
        in_specs=[
            pl.BlockSpec((B, 1, H, D), lambda c: (0, 0, 0, 0)),
            pl.BlockSpec((CHUNK, BS, H, D), lambda c: (c, 0, 0, 0)),
            pl.BlockSpec((CHUNK, BS, H, D), lambda c: (c, 0, 0, 0)),
            pl.BlockSpec((B, CTOK), lambda c: (0, c)),
        ],
        out_specs=pl.BlockSpec((B, 1, H, D), lambda c: (0, 0, 0, 0)),
        scratch_shapes=[
            pltpu.VMEM((B, H), jnp.float32),
            pltpu.VMEM((B, H), jnp.float32),
            pltpu.VMEM((B, H, D), jnp.float32),
            pltpu.VMEM((B, H), jnp.float32),
            pltpu.VMEM((B, H), jnp.float32),
            pltpu.VMEM((B, H, D), jnp.float32),
            pltpu.SemaphoreType.DMA((3,)),
            pltpu.SemaphoreType.DMA((3,)),
        ],
        compiler_params=pltpu.CompilerParams(
            dimension_semantics=("arbitrary",),
            collective_id=0,
        ),
    )(Q, K, V, w)
